# Initial kernel scaffold; baseline (speedup 1.0000x reference)
import jax
import jax.numpy as jnp
from jax.experimental import pallas as pl


def kernel(x, edge_index, edge_type, batch_size, W_rel, W_self, b, W_lin, b_lin):
    raise NotImplementedError("write your pallas kernel here")


# speedup vs baseline: 11.9953x; 11.9953x over previous
import functools

import jax
import jax.numpy as jnp
from jax import lax
from jax.experimental import pallas as pl
from jax.experimental.pallas import tpu as pltpu
from jax.experimental.pallas import tpu_sc as plsc

_NC = 2
_NS = 16
_L = 16
_C = 80


def _matmul_body(x_ref, w_ref, o_ref):
    o_ref[0] = jnp.dot(x_ref[...], w_ref[0], preferred_element_type=jnp.float32)


def _project_all(x, Wcat):
    N, D = x.shape
    RR, _, H = Wcat.shape
    BN = 1000
    grid = (N // BN, RR)
    return pl.pallas_call(
        _matmul_body,
        grid=grid,
        in_specs=[
            pl.BlockSpec((BN, D), lambda i, r: (i, 0)),
            pl.BlockSpec((1, D, H), lambda i, r: (r, 0, 0)),
        ],
        out_specs=pl.BlockSpec((1, BN, H), lambda i, r: (r, i, 0)),
        out_shape=jax.ShapeDtypeStruct((RR, N, H), jnp.float32),
    )(x, Wcat)


def _head_body(h0_ref, p_ref, b_ref, wl_ref, o_ref):
    s = h0_ref[0] + p_ref[0, 0] + p_ref[1, 0] + b_ref[...]
    s = jnp.maximum(s, 0.0)
    m = jnp.sum(s, axis=0, keepdims=True)
    o_ref[0] = jnp.dot(m, wl_ref[...], preferred_element_type=jnp.float32)


def _pool_head(h0r, Pr, b, W_lin):
    B, PG, H = h0r.shape
    ND = W_lin.shape[1]
    return pl.pallas_call(
        _head_body,
        grid=(B,),
        in_specs=[
            pl.BlockSpec((1, PG, H), lambda g: (g, 0, 0)),
            pl.BlockSpec((2, 1, PG, H), lambda g: (0, g, 0, 0)),
            pl.BlockSpec((1, H), lambda g: (0, 0)),
            pl.BlockSpec((H, ND), lambda g: (0, 0)),
        ],
        out_specs=pl.BlockSpec((1, 1, ND), lambda g: (g, 0, 0)),
        out_shape=jax.ShapeDtypeStruct((B, 1, ND), jnp.float32),
    )(h0r, Pr, b.reshape(1, H), W_lin).reshape(B, ND)


def _make_sc_edge_kernel(N, E, H, R):
    NK = N * R
    kslice = -(-NK // (_NS * 8)) * 8
    NKP = kslice * _NS
    ept = E // (_NC * _NS)
    eph = E // _NS
    rpt = -(-(N // _NS) // 8) * 8
    NP = rpt * _NS
    mesh = plsc.VectorSubcoreMesh(core_axis_name="c", subcore_axis_name="s")

    @functools.partial(
        pl.kernel,
        out_type=jax.ShapeDtypeStruct((_NC, NP, H), jnp.float32),
        mesh=mesh,
        scratch_types=[
            pltpu.VMEM((_C,), jnp.int32),
            pltpu.VMEM((_C,), jnp.int32),
            pltpu.VMEM((_C,), jnp.int32),
            pltpu.VMEM((_C,), jnp.int32),
            pltpu.VMEM((_C,), jnp.int32),
            pltpu.VMEM((_C,), jnp.float32),
            pltpu.VMEM((_C,), jnp.float32),
            pltpu.VMEM((kslice,), jnp.float32),
            pltpu.VMEM((_C, H), jnp.float32),
            pltpu.VMEM_SHARED((NKP,), jnp.float32),
            pltpu.VMEM_SHARED((NP, H), jnp.float32),
            pltpu.SemaphoreType.DMA,
        ],
        compiler_params=pltpu.CompilerParams(needs_layout_passes=False),
    )
    def sc_kernel(y_hbm, src_hbm, dst_hbm, et_hbm, zc_hbm, zr_hbm, out_hbm,
                  srcb, dstb, etb, gidxb, kidxb, wb, onesb, recb, rowsb,
                  cnt_sh, hacc_sh, sem):
        cid = lax.axis_index("c")
        sid = lax.axis_index("s")

        for g in range(_C // _L):
            onesb[pl.ds(g * _L, _L)] = jnp.ones((_L,), jnp.float32)

        pltpu.sync_copy(zc_hbm, cnt_sh.at[pl.ds(sid * kslice, kslice)])
        pltpu.sync_copy(zr_hbm, hacc_sh.at[pl.ds(sid * rpt, rpt)])
        plsc.subcore_barrier()

        def hist_body(i, carry):
            base = sid * eph + i * _C
            pltpu.sync_copy(dst_hbm.at[pl.ds(base, _C)], dstb)
            pltpu.sync_copy(et_hbm.at[pl.ds(base, _C)], etb)
            for g in range(_C // _L):
                s = pl.ds(g * _L, _L)
                kidxb[s] = dstb[s] * R + etb[s]
            pltpu.sync_copy(onesb, cnt_sh.at[kidxb], add=True)
            return carry
        lax.fori_loop(0, eph // _C, hist_body, 0)
        plsc.subcore_barrier()

        pltpu.sync_copy(cnt_sh.at[pl.ds(sid * kslice, kslice)], recb)
        def rec_body(i, carry):
            s = pl.ds(i * _L, _L)
            recb[s] = 1.0 / jnp.maximum(recb[s], 1.0)
            return carry
        lax.fori_loop(0, kslice // _L, rec_body, 0)
        pltpu.sync_copy(recb, cnt_sh.at[pl.ds(sid * kslice, kslice)])
        plsc.subcore_barrier()

        def edge_body(i, carry):
            base = cid * (E // _NC) + sid * ept + i * _C
            pltpu.sync_copy(src_hbm.at[pl.ds(base, _C)], srcb)
            pltpu.sync_copy(dst_hbm.at[pl.ds(base, _C)], dstb)
            pltpu.sync_copy(et_hbm.at[pl.ds(base, _C)], etb)
            for g in range(_C // _L):
                s = pl.ds(g * _L, _L)
                kidxb[s] = dstb[s] * R + etb[s]
                gidxb[s] = etb[s] * N + srcb[s]
            pltpu.sync_copy(cnt_sh.at[kidxb], wb)
            pltpu.async_copy(y_hbm.at[gidxb], rowsb, sem).wait()

            def scale_body(e, c2):
                w16 = plsc.load_gather(wb, [jnp.full((_L,), e, jnp.int32)])
                for j in range(H // _L):
                    sj = pl.ds(j * _L, _L)
                    rowsb[e, sj] = rowsb[e, sj] * w16
                return c2
            lax.fori_loop(0, _C, scale_body, 0)
            pltpu.sync_copy(rowsb, hacc_sh.at[dstb], add=True)
            return carry
        lax.fori_loop(0, ept // _C, edge_body, 0)
        plsc.subcore_barrier()

        pltpu.sync_copy(hacc_sh.at[pl.ds(sid * rpt, rpt)],
                        out_hbm.at[cid, pl.ds(sid * rpt, rpt)])

    return sc_kernel


def kernel(x, edge_index, edge_type, batch_size, W_rel, W_self, b, W_lin, b_lin):
    N, D = x.shape
    R, _, H = W_rel.shape
    E = edge_index.shape[1]
    ND = W_lin.shape[1]
    B = 16

    Wcat = jnp.concatenate([W_rel, W_self[None]], axis=0)
    Y = _project_all(x, Wcat)
    Yflat = Y.reshape((R + 1) * N, H)

    src = edge_index[0].astype(jnp.int32)
    dst = edge_index[1].astype(jnp.int32)
    et = edge_type.astype(jnp.int32)

    sc = _make_sc_edge_kernel(N, E, H, R)
    kslice = -(-(N * R) // (_NS * 8)) * 8
    rpt = -(-(N // _NS) // 8) * 8
    zc = jnp.zeros((kslice,), jnp.float32)
    zr = jnp.zeros((rpt, H), jnp.float32)
    P = sc(Yflat, src, dst, et, zc, zr)[:, :N, :]

    h0r = Y[R].reshape(B, N // B, H)
    Pr = P.reshape(2, B, N // B, H)
    osum = _pool_head(h0r, Pr, b, W_lin)
    per_graph = N // batch_size
    out = osum / per_graph.astype(jnp.float32) + b_lin[None, :]
    return jnp.squeeze(out)


# speedup vs baseline: 14.3223x; 1.1940x over previous
import functools

import jax
import jax.numpy as jnp
from jax import lax
from jax.experimental import pallas as pl
from jax.experimental.pallas import tpu as pltpu
from jax.experimental.pallas import tpu_sc as plsc

_NC = 2
_NS = 16
_L = 16
_C = 80


def _matmul_body(x_ref, w_ref, o_ref):
    o_ref[0] = jnp.dot(x_ref[...], w_ref[0], preferred_element_type=jnp.float32)


def _project_all(x, Wcat):
    N, D = x.shape
    RR, _, H = Wcat.shape
    BN = 1000
    return pl.pallas_call(
        _matmul_body,
        grid=(N // BN, RR),
        in_specs=[
            pl.BlockSpec((BN, D), lambda i, r: (i, 0)),
            pl.BlockSpec((1, D, H), lambda i, r: (r, 0, 0)),
        ],
        out_specs=pl.BlockSpec((1, BN, H), lambda i, r: (r, i, 0)),
        out_shape=jax.ShapeDtypeStruct((RR, N, H), jnp.float32),
    )(x, Wcat)


def _head_body(h0_ref, p_ref, b_ref, wl_ref, o_ref):
    s = h0_ref[0] + p_ref[0, 0] + p_ref[1, 0] + b_ref[...]
    s = jnp.maximum(s, 0.0)
    m = jnp.sum(s, axis=0, keepdims=True)
    o_ref[0] = jnp.dot(m, wl_ref[...], preferred_element_type=jnp.float32)


def _pool_head(h0r, Pr, b, W_lin):
    B, PG, H = h0r.shape
    ND = W_lin.shape[1]
    return pl.pallas_call(
        _head_body,
        grid=(B,),
        in_specs=[
            pl.BlockSpec((1, PG, H), lambda g: (g, 0, 0)),
            pl.BlockSpec((2, 1, PG, H), lambda g: (0, g, 0, 0)),
            pl.BlockSpec((1, H), lambda g: (0, 0)),
            pl.BlockSpec((H, ND), lambda g: (0, 0)),
        ],
        out_specs=pl.BlockSpec((1, 1, ND), lambda g: (g, 0, 0)),
        out_shape=jax.ShapeDtypeStruct((B, 1, ND), jnp.float32),
    )(h0r, Pr, b.reshape(1, H), W_lin).reshape(B, ND)


def _make_sc_edge_kernel(N, E, H, R):
    NK = N * R
    kslice = -(-NK // (_NS * 8)) * 8
    NKP = kslice * _NS
    ept = E // (_NC * _NS)
    eph = E // _NS
    rpt = -(-(N // _NS) // 8) * 8
    NP = rpt * _NS
    NCH = ept // _C
    NCHH = eph // _C
    GR = _C // _L
    mesh = plsc.VectorSubcoreMesh(core_axis_name="c", subcore_axis_name="s")

    @functools.partial(
        pl.kernel,
        out_type=jax.ShapeDtypeStruct((_NC, NP, H), jnp.float32),
        mesh=mesh,
        scratch_types=[
            [pltpu.VMEM((_C,), jnp.int32)] * 2,
            [pltpu.VMEM((_C,), jnp.int32)] * 2,
            [pltpu.VMEM((_C,), jnp.int32)] * 2,
            [pltpu.VMEM((_C,), jnp.float32)] * 2,
            [pltpu.VMEM((_C, H), jnp.float32)] * 2,
            pltpu.VMEM((_C,), jnp.float32),
            pltpu.VMEM((kslice,), jnp.float32),
            pltpu.VMEM_SHARED((NKP,), jnp.float32),
            pltpu.VMEM_SHARED((NP, H), jnp.float32),
            [pltpu.SemaphoreType.DMA] * 2,
        ],
        compiler_params=pltpu.CompilerParams(needs_layout_passes=False),
    )
    def sc_kernel(y_hbm, src_hbm, dst_hbm, et_hbm, zc_hbm, zr_hbm, out_hbm,
                  srcb, dstb, etb, wb, rows, onesb, recb, cnt_sh, hacc_sh,
                  gsems):
        cid = lax.axis_index("c")
        sid = lax.axis_index("s")

        for g in range(GR):
            onesb[pl.ds(g * _L, _L)] = jnp.ones((_L,), jnp.float32)

        pltpu.sync_copy(zc_hbm, cnt_sh.at[pl.ds(sid * kslice, kslice)])
        pltpu.sync_copy(zr_hbm, hacc_sh.at[pl.ds(sid * rpt, rpt)])
        plsc.subcore_barrier()

        def hist_load(i, k):
            base = sid * eph + i * _C
            pltpu.sync_copy(dst_hbm.at[pl.ds(base, _C)], dstb[k])
            pltpu.sync_copy(et_hbm.at[pl.ds(base, _C)], etb[k])
            for g in range(GR):
                s = pl.ds(g * _L, _L)
                etb[k][s] = dstb[k][s] * R + etb[k][s]
            return pltpu.async_copy(onesb, cnt_sh.at[etb[k]], gsems[k],
                                    add=True)

        def hist_body(p, carry):
            d0 = hist_load(2 * p, 0)
            d1 = hist_load(2 * p + 1, 1)
            d0.wait()
            d1.wait()
            return carry
        lax.fori_loop(0, NCHH // 2, hist_body, 0)
        plsc.subcore_barrier()

        pltpu.sync_copy(cnt_sh.at[pl.ds(sid * kslice, kslice)], recb)

        def rec_body(i, carry):
            s = pl.ds(i * _L, _L)
            recb[s] = 1.0 / jnp.maximum(recb[s], 1.0)
            return carry
        lax.fori_loop(0, kslice // _L, rec_body, 0)
        pltpu.sync_copy(recb, cnt_sh.at[pl.ds(sid * kslice, kslice)])
        plsc.subcore_barrier()

        ebase = cid * (E // _NC) + sid * ept

        def edge_load(i, k):
            base = ebase + i * _C
            pltpu.sync_copy(src_hbm.at[pl.ds(base, _C)], srcb[k])
            pltpu.sync_copy(dst_hbm.at[pl.ds(base, _C)], dstb[k])
            pltpu.sync_copy(et_hbm.at[pl.ds(base, _C)], etb[k])
            for g in range(GR):
                s = pl.ds(g * _L, _L)
                srcb[k][s] = etb[k][s] * N + srcb[k][s]
                etb[k][s] = dstb[k][s] * R + etb[k][s]
            pltpu.sync_copy(cnt_sh.at[etb[k]], wb[k])
            return pltpu.async_copy(y_hbm.at[srcb[k]], rows[k], gsems[k])

        def edge_compute(k):
            def scale_body(e, c2):
                w16 = plsc.load_gather(wb[k],
                                       [jnp.full((_L,), e, jnp.int32)])
                for j in range(H // _L):
                    sj = pl.ds(j * _L, _L)
                    rows[k][e, sj] = rows[k][e, sj] * w16
                return c2
            lax.fori_loop(0, _C, scale_body, 0)
            pltpu.sync_copy(rows[k], hacc_sh.at[dstb[k]], add=True)

        def edge_body(p, carry):
            d0 = edge_load(2 * p, 0)
            d1 = edge_load(2 * p + 1, 1)
            d0.wait()
            edge_compute(0)
            d1.wait()
            edge_compute(1)
            return carry
        lax.fori_loop(0, NCH // 2, edge_body, 0)
        edge_load(NCH - 1, 0).wait()
        edge_compute(0)
        plsc.subcore_barrier()

        pltpu.sync_copy(hacc_sh.at[pl.ds(sid * rpt, rpt)],
                        out_hbm.at[cid, pl.ds(sid * rpt, rpt)])

    return sc_kernel


def kernel(x, edge_index, edge_type, batch_size, W_rel, W_self, b, W_lin, b_lin):
    N, D = x.shape
    R, _, H = W_rel.shape
    E = edge_index.shape[1]
    B = 16

    Wcat = jnp.concatenate([W_rel, W_self[None]], axis=0)
    Y = _project_all(x, Wcat)
    Yflat = Y.reshape((R + 1) * N, H)

    src = edge_index[0].astype(jnp.int32)
    dst = edge_index[1].astype(jnp.int32)
    et = edge_type.astype(jnp.int32)

    sc = _make_sc_edge_kernel(N, E, H, R)
    kslice = -(-(N * R) // (_NS * 8)) * 8
    rpt = -(-(N // _NS) // 8) * 8
    zc = jnp.zeros((kslice,), jnp.float32)
    zr = jnp.zeros((rpt, H), jnp.float32)
    P = sc(Yflat, src, dst, et, zc, zr)[:, :N, :]

    h0r = Y[R].reshape(B, N // B, H)
    Pr = P.reshape(2, B, N // B, H)
    osum = _pool_head(h0r, Pr, b, W_lin)
    per_graph = N // batch_size
    out = osum / per_graph.astype(jnp.float32) + b_lin[None, :]
    return jnp.squeeze(out)


# speedup vs baseline: 23.0824x; 1.6116x over previous
import functools

import jax
import jax.numpy as jnp
from jax import lax
from jax.experimental import pallas as pl
from jax.experimental.pallas import tpu as pltpu
from jax.experimental.pallas import tpu_sc as plsc

_NC = 2
_NS = 16
_L = 16
_C = 80


def _matmul_body(x_ref, w_ref, o_ref):
    o_ref[0] = jnp.dot(x_ref[...], w_ref[0], preferred_element_type=jnp.float32)


def _project_all(x, Wcat):
    N, D = x.shape
    RR, _, H = Wcat.shape
    BN = 1000
    return pl.pallas_call(
        _matmul_body,
        grid=(N // BN, RR),
        in_specs=[
            pl.BlockSpec((BN, D), lambda i, r: (i, 0)),
            pl.BlockSpec((1, D, H), lambda i, r: (r, 0, 0)),
        ],
        out_specs=pl.BlockSpec((1, BN, H), lambda i, r: (r, i, 0)),
        out_shape=jax.ShapeDtypeStruct((RR, N, H), jnp.float32),
    )(x, Wcat)


def _head_body(h0_ref, p_ref, b_ref, wl_ref, o_ref):
    s = h0_ref[0] + p_ref[0, 0] + p_ref[1, 0] + b_ref[...]
    s = jnp.maximum(s, 0.0)
    m = jnp.sum(s, axis=0, keepdims=True)
    o_ref[0] = jnp.dot(m, wl_ref[...], preferred_element_type=jnp.float32)


def _pool_head(h0r, Pr, b, W_lin):
    B, PG, H = h0r.shape
    ND = W_lin.shape[1]
    return pl.pallas_call(
        _head_body,
        grid=(B,),
        in_specs=[
            pl.BlockSpec((1, PG, H), lambda g: (g, 0, 0)),
            pl.BlockSpec((2, 1, PG, H), lambda g: (0, g, 0, 0)),
            pl.BlockSpec((1, H), lambda g: (0, 0)),
            pl.BlockSpec((H, ND), lambda g: (0, 0)),
        ],
        out_specs=pl.BlockSpec((1, 1, ND), lambda g: (g, 0, 0)),
        out_shape=jax.ShapeDtypeStruct((B, 1, ND), jnp.float32),
    )(h0r, Pr, b.reshape(1, H), W_lin).reshape(B, ND)


def _make_sc_edge_kernel(N, E, H, R):
    NK = N * R
    kslice = -(-NK // (_NS * 8)) * 8
    NKP = kslice * _NS
    ept = E // (_NC * _NS)
    eph = E // _NS
    rpt = -(-(N // _NS) // 8) * 8
    NP = rpt * _NS
    NCH = ept // _C
    NCHH = eph // _C
    GR = _C // _L
    mesh = plsc.VectorSubcoreMesh(core_axis_name="c", subcore_axis_name="s")

    SB = 25
    SE = SB * _C

    @functools.partial(
        pl.kernel,
        out_type=jax.ShapeDtypeStruct((_NC, NP, H), jnp.float32),
        mesh=mesh,
        scratch_types=[
            pltpu.VMEM((SE,), jnp.int32),
            pltpu.VMEM((SE,), jnp.int32),
            pltpu.VMEM((SE,), jnp.int32),
            [pltpu.VMEM((_C,), jnp.int32)] * 2,
            [pltpu.VMEM((_C,), jnp.int32)] * 2,
            [pltpu.VMEM((_C,), jnp.int32)] * 2,
            [pltpu.VMEM((_C,), jnp.float32)] * 2,
            [pltpu.VMEM((_C, H), jnp.float32)] * 2,
            pltpu.VMEM((_C,), jnp.float32),
            pltpu.VMEM((kslice,), jnp.float32),
            pltpu.VMEM_SHARED((NKP,), jnp.float32),
            pltpu.VMEM_SHARED((NP, H), jnp.float32),
            [pltpu.SemaphoreType.DMA] * 2,
            [pltpu.SemaphoreType.DMA] * 2,
        ],
        compiler_params=pltpu.CompilerParams(needs_layout_passes=False),
    )
    def sc_kernel(y_hbm, src_hbm, dst_hbm, et_hbm, zc_hbm, zr_hbm, out_hbm,
                  src1, dst1, et1, sib, gib, kib, wb, rows, onesb, recb,
                  cnt_sh, hacc_sh, gsems, ssems):
        cid = lax.axis_index("c")
        sid = lax.axis_index("s")

        for g in range(GR):
            onesb[pl.ds(g * _L, _L)] = jnp.ones((_L,), jnp.float32)

        pltpu.sync_copy(zc_hbm, cnt_sh.at[pl.ds(sid * kslice, kslice)])
        pltpu.sync_copy(zr_hbm, hacc_sh.at[pl.ds(sid * rpt, rpt)])
        plsc.subcore_barrier()

        def hist_super(sc, carry):
            base = sid * eph + sc * SE
            pltpu.sync_copy(dst_hbm.at[pl.ds(base, SE)], dst1)
            pltpu.sync_copy(et_hbm.at[pl.ds(base, SE)], et1)

            def hkey_body(i, c2):
                s = pl.ds(i * _L, _L)
                et1[s] = dst1[s] * R + et1[s]
                return c2
            lax.fori_loop(0, SE // _L, hkey_body, 0)

            def hpair_body(p, c2):
                ds = []
                for k in range(2):
                    off = (2 * p + k) * _C
                    for g in range(GR):
                        sib[k][pl.ds(g * _L, _L)] = et1[pl.ds(off + g * _L,
                                                              _L)]
                    ds.append(pltpu.async_copy(onesb, cnt_sh.at[sib[k]],
                                               gsems[k], add=True))
                ds[0].wait()
                ds[1].wait()
                return c2
            lax.fori_loop(0, SB // 2, hpair_body, 0)
            off = (SB - 1) * _C
            for g in range(GR):
                sib[0][pl.ds(g * _L, _L)] = et1[pl.ds(off + g * _L, _L)]
            pltpu.async_copy(onesb, cnt_sh.at[sib[0]], gsems[0],
                             add=True).wait()
            return carry
        lax.fori_loop(0, eph // SE, hist_super, 0)
        plsc.subcore_barrier()

        pltpu.sync_copy(cnt_sh.at[pl.ds(sid * kslice, kslice)], recb)

        def rec_body(i, carry):
            s = pl.ds(i * _L, _L)
            recb[s] = 1.0 / jnp.maximum(recb[s], 1.0)
            return carry
        lax.fori_loop(0, kslice // _L, rec_body, 0)
        pltpu.sync_copy(recb, cnt_sh.at[pl.ds(sid * kslice, kslice)])
        plsc.subcore_barrier()

        ebase = cid * (E // _NC) + sid * ept

        def eload(i, k):
            off = i * _C
            for g in range(GR):
                s = pl.ds(g * _L, _L)
                so = pl.ds(off + g * _L, _L)
                sib[k][s] = dst1[so]
                gib[k][s] = src1[so]
                kib[k][s] = et1[so]
            pltpu.sync_copy(cnt_sh.at[kib[k]], wb[k])
            return pltpu.async_copy(y_hbm.at[gib[k]], rows[k], gsems[k])

        def edge_compute(k):
            def scale_body(e2, c2):
                for u in range(2):
                    e = e2 * 2 + u
                    w16 = plsc.load_gather(wb[k],
                                           [jnp.full((_L,), e, jnp.int32)])
                    for j in range(H // _L):
                        sj = pl.ds(j * _L, _L)
                        rows[k][e, sj] = rows[k][e, sj] * w16
                return c2
            lax.fori_loop(0, _C // 2, scale_body, 0)
            pltpu.sync_copy(rows[k], hacc_sh.at[sib[k]], add=True)

        def edge_super(sc, carry):
            base = ebase + sc * SE
            pltpu.sync_copy(src_hbm.at[pl.ds(base, SE)], src1)
            pltpu.sync_copy(dst_hbm.at[pl.ds(base, SE)], dst1)
            pltpu.sync_copy(et_hbm.at[pl.ds(base, SE)], et1)

            def ekey_body(i, c2):
                s = pl.ds(i * _L, _L)
                src1[s] = et1[s] * N + src1[s]
                et1[s] = dst1[s] * R + et1[s]
                return c2
            lax.fori_loop(0, SE // _L, ekey_body, 0)

            def epair_body(p, c2):
                dr0 = eload(2 * p, 0)
                dr1 = eload(2 * p + 1, 1)
                dr0.wait()
                edge_compute(0)
                dr1.wait()
                edge_compute(1)
                return c2
            lax.fori_loop(0, SB // 2, epair_body, 0)
            eload(SB - 1, 0).wait()
            edge_compute(0)
            return carry
        lax.fori_loop(0, ept // SE, edge_super, 0)
        plsc.subcore_barrier()

        pltpu.sync_copy(hacc_sh.at[pl.ds(sid * rpt, rpt)],
                        out_hbm.at[cid, pl.ds(sid * rpt, rpt)])

    return sc_kernel


def kernel(x, edge_index, edge_type, batch_size, W_rel, W_self, b, W_lin, b_lin):
    N, D = x.shape
    R, _, H = W_rel.shape
    E = edge_index.shape[1]
    B = 16

    Wcat = jnp.concatenate([W_rel, W_self[None]], axis=0)
    Y = _project_all(x, Wcat)
    Yflat = Y.reshape((R + 1) * N, H)

    src = edge_index[0].astype(jnp.int32)
    dst = edge_index[1].astype(jnp.int32)
    et = edge_type.astype(jnp.int32)

    sc = _make_sc_edge_kernel(N, E, H, R)
    kslice = -(-(N * R) // (_NS * 8)) * 8
    rpt = -(-(N // _NS) // 8) * 8
    zc = jnp.zeros((kslice,), jnp.float32)
    zr = jnp.zeros((rpt, H), jnp.float32)
    P = sc(Yflat, src, dst, et, zc, zr)[:, :N, :]

    h0r = Y[R].reshape(B, N // B, H)
    Pr = P.reshape(2, B, N // B, H)
    osum = _pool_head(h0r, Pr, b, W_lin)
    per_graph = N // batch_size
    out = osum / per_graph.astype(jnp.float32) + b_lin[None, :]
    return jnp.squeeze(out)


# speedup vs baseline: 23.4011x; 1.0138x over previous
import functools

import jax
import jax.numpy as jnp
from jax import lax
from jax.experimental import pallas as pl
from jax.experimental.pallas import tpu as pltpu
from jax.experimental.pallas import tpu_sc as plsc

_NC = 2
_NS = 16
_L = 16
_C = 80


def _matmul_body(x_ref, w_ref, o_ref):
    o_ref[0] = jnp.dot(x_ref[...], w_ref[0], preferred_element_type=jnp.float32)


def _project_all(x, Wcat):
    N, D = x.shape
    RR, _, H = Wcat.shape
    BN = 1000
    return pl.pallas_call(
        _matmul_body,
        grid=(N // BN, RR),
        in_specs=[
            pl.BlockSpec((BN, D), lambda i, r: (i, 0)),
            pl.BlockSpec((1, D, H), lambda i, r: (r, 0, 0)),
        ],
        out_specs=pl.BlockSpec((1, BN, H), lambda i, r: (r, i, 0)),
        out_shape=jax.ShapeDtypeStruct((RR, N, H), jnp.float32),
    )(x, Wcat)


def _head_body(h0_ref, p_ref, b_ref, wl_ref, o_ref):
    s = h0_ref[0] + p_ref[0, 0] + p_ref[1, 0] + b_ref[...]
    s = jnp.maximum(s, 0.0)
    m = jnp.sum(s, axis=0, keepdims=True)
    o_ref[0] = jnp.dot(m, wl_ref[...], preferred_element_type=jnp.float32)


def _pool_head(h0r, Pr, b, W_lin):
    B, PG, H = h0r.shape
    ND = W_lin.shape[1]
    return pl.pallas_call(
        _head_body,
        grid=(B,),
        in_specs=[
            pl.BlockSpec((1, PG, H), lambda g: (g, 0, 0)),
            pl.BlockSpec((2, 1, PG, H), lambda g: (0, g, 0, 0)),
            pl.BlockSpec((1, H), lambda g: (0, 0)),
            pl.BlockSpec((H, ND), lambda g: (0, 0)),
        ],
        out_specs=pl.BlockSpec((1, 1, ND), lambda g: (g, 0, 0)),
        out_shape=jax.ShapeDtypeStruct((B, 1, ND), jnp.float32),
    )(h0r, Pr, b.reshape(1, H), W_lin).reshape(B, ND)


def _make_sc_edge_kernel(N, E, H, R):
    NK = N * R
    kslice = -(-NK // (_NS * 8)) * 8
    NKP = kslice * _NS
    ept = E // (_NC * _NS)
    eph = E // _NS
    rpt = -(-(N // _NS) // 8) * 8
    NP = rpt * _NS
    NCH = ept // _C
    NCHH = eph // _C
    GR = _C // _L
    mesh = plsc.VectorSubcoreMesh(core_axis_name="c", subcore_axis_name="s")

    SB = 25
    SE = SB * _C

    @functools.partial(
        pl.kernel,
        out_type=jax.ShapeDtypeStruct((_NC, NP, H), jnp.float32),
        mesh=mesh,
        scratch_types=[
            pltpu.VMEM((SE,), jnp.int32),
            pltpu.VMEM((SE,), jnp.int32),
            pltpu.VMEM((SE,), jnp.int32),
            [pltpu.VMEM((_C,), jnp.int32)] * 2,
            [pltpu.VMEM((_C,), jnp.int32)] * 2,
            [pltpu.VMEM((_C,), jnp.int32)] * 2,
            [pltpu.VMEM((_C,), jnp.float32)] * 2,
            [pltpu.VMEM((_C, H), jnp.float32)] * 2,
            pltpu.VMEM((_C,), jnp.float32),
            pltpu.VMEM((kslice,), jnp.float32),
            pltpu.VMEM_SHARED((NKP,), jnp.float32),
            pltpu.VMEM_SHARED((NP, H), jnp.float32),
            [pltpu.SemaphoreType.DMA] * 2,
            [pltpu.SemaphoreType.DMA] * 2,
        ],
        compiler_params=pltpu.CompilerParams(needs_layout_passes=False),
    )
    def sc_kernel(y_hbm, src_hbm, dst_hbm, et_hbm, zc_hbm, zr_hbm, out_hbm,
                  src1, dst1, et1, sib, gib, kib, wb, rows, onesb, recb,
                  cnt_sh, hacc_sh, gsems, wsems):
        cid = lax.axis_index("c")
        sid = lax.axis_index("s")

        for g in range(GR):
            onesb[pl.ds(g * _L, _L)] = jnp.ones((_L,), jnp.float32)

        pltpu.sync_copy(zc_hbm, cnt_sh.at[pl.ds(sid * kslice, kslice)])
        pltpu.sync_copy(zr_hbm, hacc_sh.at[pl.ds(sid * rpt, rpt)])
        plsc.subcore_barrier()

        def hist_super(sc, carry):
            base = sid * eph + sc * SE
            pltpu.sync_copy(dst_hbm.at[pl.ds(base, SE)], dst1)
            pltpu.sync_copy(et_hbm.at[pl.ds(base, SE)], et1)

            def hkey_body(i, c2):
                s = pl.ds(i * _L, _L)
                et1[s] = dst1[s] * R + et1[s]
                return c2
            lax.fori_loop(0, SE // _L, hkey_body, 0)

            def hpair_body(p, c2):
                ds = []
                for k in range(2):
                    off = (2 * p + k) * _C
                    for g in range(GR):
                        sib[k][pl.ds(g * _L, _L)] = et1[pl.ds(off + g * _L,
                                                              _L)]
                    ds.append(pltpu.async_copy(onesb, cnt_sh.at[sib[k]],
                                               gsems[k], add=True))
                ds[0].wait()
                ds[1].wait()
                return c2
            lax.fori_loop(0, SB // 2, hpair_body, 0)
            off = (SB - 1) * _C
            for g in range(GR):
                sib[0][pl.ds(g * _L, _L)] = et1[pl.ds(off + g * _L, _L)]
            pltpu.async_copy(onesb, cnt_sh.at[sib[0]], gsems[0],
                             add=True).wait()
            return carry
        lax.fori_loop(0, eph // SE, hist_super, 0)
        plsc.subcore_barrier()

        pltpu.sync_copy(cnt_sh.at[pl.ds(sid * kslice, kslice)], recb)

        def rec_body(i, carry):
            s = pl.ds(i * _L, _L)
            recb[s] = 1.0 / jnp.maximum(recb[s], 1.0)
            return carry
        lax.fori_loop(0, kslice // _L, rec_body, 0)
        pltpu.sync_copy(recb, cnt_sh.at[pl.ds(sid * kslice, kslice)])
        plsc.subcore_barrier()

        ebase = cid * (E // _NC) + sid * ept

        def eload(i, k):
            off = i * _C
            for g in range(GR):
                s = pl.ds(g * _L, _L)
                so = pl.ds(off + g * _L, _L)
                sib[k][s] = dst1[so]
                gib[k][s] = src1[so]
                kib[k][s] = et1[so]
            dw = pltpu.async_copy(cnt_sh.at[kib[k]], wb[k], wsems[k])
            dr = pltpu.async_copy(y_hbm.at[gib[k]], rows[k], gsems[k])
            return dw, dr

        def edge_compute(k):
            def scale_body(e4, c2):
                for u in range(4):
                    e = e4 * 4 + u
                    w16 = plsc.load_gather(wb[k],
                                           [jnp.full((_L,), e, jnp.int32)])
                    for j in range(H // _L):
                        sj = pl.ds(j * _L, _L)
                        rows[k][e, sj] = rows[k][e, sj] * w16
                return c2
            lax.fori_loop(0, _C // 4, scale_body, 0)
            pltpu.sync_copy(rows[k], hacc_sh.at[sib[k]], add=True)

        def edge_super(sc, carry):
            base = ebase + sc * SE
            pltpu.sync_copy(src_hbm.at[pl.ds(base, SE)], src1)
            pltpu.sync_copy(dst_hbm.at[pl.ds(base, SE)], dst1)
            pltpu.sync_copy(et_hbm.at[pl.ds(base, SE)], et1)

            def ekey_body(i, c2):
                s = pl.ds(i * _L, _L)
                src1[s] = et1[s] * N + src1[s]
                et1[s] = dst1[s] * R + et1[s]
                return c2
            lax.fori_loop(0, SE // _L, ekey_body, 0)

            def epair_body(p, c2):
                dw0, dr0 = eload(2 * p, 0)
                dw1, dr1 = eload(2 * p + 1, 1)
                dw0.wait()
                dr0.wait()
                edge_compute(0)
                dw1.wait()
                dr1.wait()
                edge_compute(1)
                return c2
            lax.fori_loop(0, SB // 2, epair_body, 0)
            dw, dr = eload(SB - 1, 0)
            dw.wait()
            dr.wait()
            edge_compute(0)
            return carry
        lax.fori_loop(0, ept // SE, edge_super, 0)
        plsc.subcore_barrier()

        pltpu.sync_copy(hacc_sh.at[pl.ds(sid * rpt, rpt)],
                        out_hbm.at[cid, pl.ds(sid * rpt, rpt)])

    return sc_kernel


def kernel(x, edge_index, edge_type, batch_size, W_rel, W_self, b, W_lin, b_lin):
    N, D = x.shape
    R, _, H = W_rel.shape
    E = edge_index.shape[1]
    B = 16

    Wcat = jnp.concatenate([W_rel, W_self[None]], axis=0)
    Y = _project_all(x, Wcat)
    Yflat = Y.reshape((R + 1) * N, H)

    src = edge_index[0].astype(jnp.int32)
    dst = edge_index[1].astype(jnp.int32)
    et = edge_type.astype(jnp.int32)

    sc = _make_sc_edge_kernel(N, E, H, R)
    kslice = -(-(N * R) // (_NS * 8)) * 8
    rpt = -(-(N // _NS) // 8) * 8
    zc = jnp.zeros((kslice,), jnp.float32)
    zr = jnp.zeros((rpt, H), jnp.float32)
    P = sc(Yflat, src, dst, et, zc, zr)[:, :N, :]

    h0r = Y[R].reshape(B, N // B, H)
    Pr = P.reshape(2, B, N // B, H)
    osum = _pool_head(h0r, Pr, b, W_lin)
    per_graph = N // batch_size
    out = osum / per_graph.astype(jnp.float32) + b_lin[None, :]
    return jnp.squeeze(out)


# speedup vs baseline: 25.0754x; 1.0715x over previous
import functools

import jax
import jax.numpy as jnp
from jax import lax
from jax.experimental import pallas as pl
from jax.experimental.pallas import tpu as pltpu
from jax.experimental.pallas import tpu_sc as plsc

_NC = 2
_NS = 16
_L = 16
_C = 80


def _matmul_body(x_ref, w_ref, o_ref):
    o_ref[0] = jnp.dot(x_ref[...], w_ref[0], preferred_element_type=jnp.float32)


def _project_all(x, Wcat):
    N, D = x.shape
    RR, _, H = Wcat.shape
    BN = 1000
    return pl.pallas_call(
        _matmul_body,
        grid=(N // BN, RR),
        in_specs=[
            pl.BlockSpec((BN, D), lambda i, r: (i, 0)),
            pl.BlockSpec((1, D, H), lambda i, r: (r, 0, 0)),
        ],
        out_specs=pl.BlockSpec((1, BN, H), lambda i, r: (r, i, 0)),
        out_shape=jax.ShapeDtypeStruct((RR, N, H), jnp.float32),
    )(x, Wcat)


def _head_body(h0_ref, p_ref, b_ref, wl_ref, o_ref):
    s = h0_ref[0] + p_ref[0, 0] + p_ref[1, 0] + b_ref[...]
    s = jnp.maximum(s, 0.0)
    m = jnp.sum(s, axis=0, keepdims=True)
    o_ref[0] = jnp.dot(m, wl_ref[...], preferred_element_type=jnp.float32)


def _pool_head(h0r, Pr, b, W_lin):
    B, PG, H = h0r.shape
    ND = W_lin.shape[1]
    return pl.pallas_call(
        _head_body,
        grid=(B,),
        in_specs=[
            pl.BlockSpec((1, PG, H), lambda g: (g, 0, 0)),
            pl.BlockSpec((2, 1, PG, H), lambda g: (0, g, 0, 0)),
            pl.BlockSpec((1, H), lambda g: (0, 0)),
            pl.BlockSpec((H, ND), lambda g: (0, 0)),
        ],
        out_specs=pl.BlockSpec((1, 1, ND), lambda g: (g, 0, 0)),
        out_shape=jax.ShapeDtypeStruct((B, 1, ND), jnp.float32),
    )(h0r, Pr, b.reshape(1, H), W_lin).reshape(B, ND)


def _make_sc_edge_kernel(N, E, H, R):
    NK = N * R
    kslice = -(-NK // (_NS * 8)) * 8
    NKP = kslice * _NS
    ept = E // (_NC * _NS)
    eph = E // _NS
    rpt = -(-(N // _NS) // 8) * 8
    NP = rpt * _NS
    NCH = ept // _C
    NCHH = eph // _C
    GR = _C // _L
    mesh = plsc.VectorSubcoreMesh(core_axis_name="c", subcore_axis_name="s")

    SB = 25
    SE = SB * _C

    @functools.partial(
        pl.kernel,
        out_type=jax.ShapeDtypeStruct((_NC, NP, H), jnp.float32),
        mesh=mesh,
        scratch_types=[
            pltpu.VMEM((SE,), jnp.int32),
            pltpu.VMEM((SE,), jnp.int32),
            pltpu.VMEM((SE,), jnp.int32),
            [pltpu.VMEM((_C,), jnp.int32)] * 4,
            [pltpu.VMEM((_C,), jnp.int32)] * 2,
            [pltpu.VMEM((_C,), jnp.int32)] * 2,
            [pltpu.VMEM((_C,), jnp.float32)] * 2,
            [pltpu.VMEM((_C, H), jnp.float32)] * 2,
            pltpu.VMEM((_C,), jnp.float32),
            pltpu.VMEM((kslice,), jnp.float32),
            pltpu.VMEM_SHARED((NKP,), jnp.float32),
            pltpu.VMEM_SHARED((NP, H), jnp.float32),
            [pltpu.SemaphoreType.DMA] * 2,
            [pltpu.SemaphoreType.DMA] * 2,
            [pltpu.SemaphoreType.DMA] * 2,
            [pltpu.SemaphoreType.DMA] * 4,
        ],
        compiler_params=pltpu.CompilerParams(needs_layout_passes=False),
    )
    def sc_kernel(y_hbm, src_hbm, dst_hbm, et_hbm, zc_hbm, zr_hbm, out_hbm,
                  src1, dst1, et1, sib, gib, kib, wb, rows, onesb, recb,
                  cnt_sh, hacc_sh, gsems, wsems, ssems, hsems):
        cid = lax.axis_index("c")
        sid = lax.axis_index("s")

        for g in range(GR):
            onesb[pl.ds(g * _L, _L)] = jnp.ones((_L,), jnp.float32)

        pltpu.sync_copy(zc_hbm, cnt_sh.at[pl.ds(sid * kslice, kslice)])
        pltpu.sync_copy(zr_hbm, hacc_sh.at[pl.ds(sid * rpt, rpt)])
        plsc.subcore_barrier()

        def hist_super(sc, carry):
            base = sid * eph + sc * SE
            pltpu.sync_copy(dst_hbm.at[pl.ds(base, SE)], dst1)
            pltpu.sync_copy(et_hbm.at[pl.ds(base, SE)], et1)

            def hkey_body(i, c2):
                s = pl.ds(i * _L, _L)
                et1[s] = dst1[s] * R + et1[s]
                return c2
            lax.fori_loop(0, SE // _L, hkey_body, 0)

            def hquad_body(p, c2):
                ds = []
                for k in range(4):
                    off = (4 * p + k) * _C
                    for g in range(GR):
                        sib[k][pl.ds(g * _L, _L)] = et1[pl.ds(off + g * _L,
                                                              _L)]
                    ds.append(pltpu.async_copy(onesb, cnt_sh.at[sib[k]],
                                               hsems[k], add=True))
                for d in ds:
                    d.wait()
                return c2
            lax.fori_loop(0, SB // 4, hquad_body, 0)
            off = (SB - 1) * _C
            for g in range(GR):
                sib[0][pl.ds(g * _L, _L)] = et1[pl.ds(off + g * _L, _L)]
            pltpu.async_copy(onesb, cnt_sh.at[sib[0]], hsems[0],
                             add=True).wait()
            return carry
        lax.fori_loop(0, eph // SE, hist_super, 0)
        plsc.subcore_barrier()

        pltpu.sync_copy(cnt_sh.at[pl.ds(sid * kslice, kslice)], recb)

        def rec_body(i, carry):
            s = pl.ds(i * _L, _L)
            recb[s] = 1.0 / jnp.maximum(recb[s], 1.0)
            return carry
        lax.fori_loop(0, kslice // _L, rec_body, 0)
        pltpu.sync_copy(recb, cnt_sh.at[pl.ds(sid * kslice, kslice)])
        plsc.subcore_barrier()

        ebase = cid * (E // _NC) + sid * ept

        def eload(i, k):
            off = i * _C
            for g in range(GR):
                s = pl.ds(g * _L, _L)
                so = pl.ds(off + g * _L, _L)
                sib[k][s] = dst1[so]
                gib[k][s] = src1[so]
                kib[k][s] = et1[so]
            dw = pltpu.async_copy(cnt_sh.at[kib[k]], wb[k], wsems[k])
            dr = pltpu.async_copy(y_hbm.at[gib[k]], rows[k], gsems[k])
            return dw, dr

        def edge_compute(k):
            def scale_body(e4, c2):
                for u in range(4):
                    e = e4 * 4 + u
                    w16 = plsc.load_gather(wb[k],
                                           [jnp.full((_L,), e, jnp.int32)])
                    for j in range(H // _L):
                        sj = pl.ds(j * _L, _L)
                        rows[k][e, sj] = rows[k][e, sj] * w16
                return c2
            lax.fori_loop(0, _C // 4, scale_body, 0)
            return pltpu.async_copy(rows[k], hacc_sh.at[sib[k]], ssems[k],
                                    add=True)

        def edge_super(sc, carry):
            base = ebase + sc * SE
            pltpu.sync_copy(src_hbm.at[pl.ds(base, SE)], src1)
            pltpu.sync_copy(dst_hbm.at[pl.ds(base, SE)], dst1)
            pltpu.sync_copy(et_hbm.at[pl.ds(base, SE)], et1)

            def ekey_body(i, c2):
                s = pl.ds(i * _L, _L)
                src1[s] = et1[s] * N + src1[s]
                et1[s] = dst1[s] * R + et1[s]
                return c2
            lax.fori_loop(0, SE // _L, ekey_body, 0)

            def epair_body(p, c2):
                dw0, dr0 = eload(2 * p, 0)
                dw1, dr1 = eload(2 * p + 1, 1)
                dw0.wait()
                dr0.wait()
                s0 = edge_compute(0)
                dw1.wait()
                dr1.wait()
                s1 = edge_compute(1)
                s0.wait()
                s1.wait()
                return c2
            lax.fori_loop(0, SB // 2, epair_body, 0)
            dw, dr = eload(SB - 1, 0)
            dw.wait()
            dr.wait()
            edge_compute(0).wait()
            return carry
        lax.fori_loop(0, ept // SE, edge_super, 0)
        plsc.subcore_barrier()

        pltpu.sync_copy(hacc_sh.at[pl.ds(sid * rpt, rpt)],
                        out_hbm.at[cid, pl.ds(sid * rpt, rpt)])

    return sc_kernel


def kernel(x, edge_index, edge_type, batch_size, W_rel, W_self, b, W_lin, b_lin):
    N, D = x.shape
    R, _, H = W_rel.shape
    E = edge_index.shape[1]
    B = 16

    Wcat = jnp.concatenate([W_rel, W_self[None]], axis=0)
    Y = _project_all(x, Wcat)
    Yflat = Y.reshape((R + 1) * N, H)

    src = edge_index[0].astype(jnp.int32)
    dst = edge_index[1].astype(jnp.int32)
    et = edge_type.astype(jnp.int32)

    sc = _make_sc_edge_kernel(N, E, H, R)
    kslice = -(-(N * R) // (_NS * 8)) * 8
    rpt = -(-(N // _NS) // 8) * 8
    zc = jnp.zeros((kslice,), jnp.float32)
    zr = jnp.zeros((rpt, H), jnp.float32)
    P = sc(Yflat, src, dst, et, zc, zr)[:, :N, :]

    h0r = Y[R].reshape(B, N // B, H)
    Pr = P.reshape(2, B, N // B, H)
    osum = _pool_head(h0r, Pr, b, W_lin)
    per_graph = N // batch_size
    out = osum / per_graph.astype(jnp.float32) + b_lin[None, :]
    return jnp.squeeze(out)


# speedup vs baseline: 26.3081x; 1.0492x over previous
import functools

import jax
import jax.numpy as jnp
from jax import lax
from jax.experimental import pallas as pl
from jax.experimental.pallas import tpu as pltpu
from jax.experimental.pallas import tpu_sc as plsc

_NC = 2
_NS = 16
_L = 16
_C = 80


def _matmul_body(x_ref, w_ref, o_ref):
    o_ref[0] = jnp.dot(x_ref[...], w_ref[0], preferred_element_type=jnp.float32)


def _project_all(x, Wcat):
    N, D = x.shape
    RR, _, H = Wcat.shape
    BN = 1000
    return pl.pallas_call(
        _matmul_body,
        grid=(N // BN, RR),
        in_specs=[
            pl.BlockSpec((BN, D), lambda i, r: (i, 0)),
            pl.BlockSpec((1, D, H), lambda i, r: (r, 0, 0)),
        ],
        out_specs=pl.BlockSpec((1, BN, H), lambda i, r: (r, i, 0)),
        out_shape=jax.ShapeDtypeStruct((RR, N, H), jnp.float32),
    )(x, Wcat)


def _head_body(h0_ref, p_ref, b_ref, wl_ref, o_ref):
    s = h0_ref[0] + p_ref[0, 0] + p_ref[1, 0] + b_ref[...]
    s = jnp.maximum(s, 0.0)
    m = jnp.sum(s, axis=0, keepdims=True)
    o_ref[0] = jnp.dot(m, wl_ref[...], preferred_element_type=jnp.float32)


def _pool_head(h0r, Pr, b, W_lin):
    B, PG, H = h0r.shape
    ND = W_lin.shape[1]
    return pl.pallas_call(
        _head_body,
        grid=(B,),
        in_specs=[
            pl.BlockSpec((1, PG, H), lambda g: (g, 0, 0)),
            pl.BlockSpec((2, 1, PG, H), lambda g: (0, g, 0, 0)),
            pl.BlockSpec((1, H), lambda g: (0, 0)),
            pl.BlockSpec((H, ND), lambda g: (0, 0)),
        ],
        out_specs=pl.BlockSpec((1, 1, ND), lambda g: (g, 0, 0)),
        out_shape=jax.ShapeDtypeStruct((B, 1, ND), jnp.float32),
    )(h0r, Pr, b.reshape(1, H), W_lin).reshape(B, ND)


def _make_sc_edge_kernel(N, E, H, R):
    NK = N * R
    kslice = -(-NK // (_NS * 8)) * 8
    NKP = kslice * _NS
    ept = E // (_NC * _NS)
    eph = E // _NS
    rpt = -(-(N // _NS) // 8) * 8
    NP = rpt * _NS
    NCH = ept // _C
    NCHH = eph // _C
    GR = _C // _L
    mesh = plsc.VectorSubcoreMesh(core_axis_name="c", subcore_axis_name="s")

    SB = 25
    SE = SB * _C

    @functools.partial(
        pl.kernel,
        out_type=jax.ShapeDtypeStruct((_NC, NP, H), jnp.float32),
        mesh=mesh,
        scratch_types=[
            pltpu.VMEM((SE,), jnp.int32),
            pltpu.VMEM((SE,), jnp.int32),
            pltpu.VMEM((SE,), jnp.int32),
            [pltpu.VMEM((_C,), jnp.int32)] * 4,
            [pltpu.VMEM((_C,), jnp.int32)] * 3,
            [pltpu.VMEM((_C,), jnp.int32)] * 3,
            [pltpu.VMEM((_C,), jnp.float32)] * 3,
            [pltpu.VMEM((_C, H), jnp.float32)] * 3,
            pltpu.VMEM((_C,), jnp.float32),
            pltpu.VMEM((kslice // 2,), jnp.float32),
            pltpu.VMEM_SHARED((NKP,), jnp.float32),
            pltpu.VMEM_SHARED((NP, H), jnp.float32),
            [pltpu.SemaphoreType.DMA] * 3,
            [pltpu.SemaphoreType.DMA] * 3,
            [pltpu.SemaphoreType.DMA] * 3,
            [pltpu.SemaphoreType.DMA] * 4,
        ],
        compiler_params=pltpu.CompilerParams(needs_layout_passes=False),
    )
    def sc_kernel(y_hbm, src_hbm, dst_hbm, et_hbm, zc_hbm, zr_hbm, out_hbm,
                  src1, dst1, et1, sib, gib, kib, wb, rows, onesb, recb,
                  cnt_sh, hacc_sh, gsems, wsems, ssems, hsems):
        cid = lax.axis_index("c")
        sid = lax.axis_index("s")

        for g in range(GR):
            onesb[pl.ds(g * _L, _L)] = jnp.ones((_L,), jnp.float32)

        pltpu.sync_copy(zc_hbm, cnt_sh.at[pl.ds(sid * kslice, kslice)])
        pltpu.sync_copy(zr_hbm, hacc_sh.at[pl.ds(sid * rpt, rpt)])
        plsc.subcore_barrier()

        def hist_super(sc, carry):
            base = sid * eph + sc * SE
            pltpu.sync_copy(dst_hbm.at[pl.ds(base, SE)], dst1)
            pltpu.sync_copy(et_hbm.at[pl.ds(base, SE)], et1)

            def hkey_body(i, c2):
                s = pl.ds(i * _L, _L)
                et1[s] = dst1[s] * R + et1[s]
                return c2
            lax.fori_loop(0, SE // _L, hkey_body, 0)

            def hquad_body(p, c2):
                ds = []
                for k in range(4):
                    off = (4 * p + k) * _C
                    for g in range(GR):
                        sib[k][pl.ds(g * _L, _L)] = et1[pl.ds(off + g * _L,
                                                              _L)]
                    ds.append(pltpu.async_copy(onesb, cnt_sh.at[sib[k]],
                                               hsems[k], add=True))
                for d in ds:
                    d.wait()
                return c2
            lax.fori_loop(0, SB // 4, hquad_body, 0)
            off = (SB - 1) * _C
            for g in range(GR):
                sib[0][pl.ds(g * _L, _L)] = et1[pl.ds(off + g * _L, _L)]
            pltpu.async_copy(onesb, cnt_sh.at[sib[0]], hsems[0],
                             add=True).wait()
            return carry
        lax.fori_loop(0, eph // SE, hist_super, 0)
        plsc.subcore_barrier()

        KH = kslice // 2
        for half in range(2):
            off = sid * kslice + half * KH
            pltpu.sync_copy(cnt_sh.at[pl.ds(off, KH)], recb)

            def rec_body(i, carry):
                s = pl.ds(i * _L, _L)
                recb[s] = 1.0 / jnp.maximum(recb[s], 1.0)
                return carry
            lax.fori_loop(0, KH // _L, rec_body, 0)
            pltpu.sync_copy(recb, cnt_sh.at[pl.ds(off, KH)])
        plsc.subcore_barrier()

        ebase = cid * (E // _NC) + sid * ept

        def eload(i, k):
            off = i * _C
            for g in range(GR):
                s = pl.ds(g * _L, _L)
                so = pl.ds(off + g * _L, _L)
                sib[k][s] = dst1[so]
                gib[k][s] = src1[so]
                kib[k][s] = et1[so]
            dw = pltpu.async_copy(cnt_sh.at[kib[k]], wb[k], wsems[k])
            dr = pltpu.async_copy(y_hbm.at[gib[k]], rows[k], gsems[k])
            return dw, dr

        def edge_compute(k):
            def scale_body(e4, c2):
                for u in range(4):
                    e = e4 * 4 + u
                    w16 = plsc.load_gather(wb[k],
                                           [jnp.full((_L,), e, jnp.int32)])
                    for j in range(H // _L):
                        sj = pl.ds(j * _L, _L)
                        rows[k][e, sj] = rows[k][e, sj] * w16
                return c2
            lax.fori_loop(0, _C // 4, scale_body, 0)
            return pltpu.async_copy(rows[k], hacc_sh.at[sib[k]], ssems[k],
                                    add=True)

        def edge_super(sc, carry):
            base = ebase + sc * SE
            pltpu.sync_copy(src_hbm.at[pl.ds(base, SE)], src1)
            pltpu.sync_copy(dst_hbm.at[pl.ds(base, SE)], dst1)
            pltpu.sync_copy(et_hbm.at[pl.ds(base, SE)], et1)

            def ekey_body(i, c2):
                s = pl.ds(i * _L, _L)
                src1[s] = et1[s] * N + src1[s]
                et1[s] = dst1[s] * R + et1[s]
                return c2
            lax.fori_loop(0, SE // _L, ekey_body, 0)

            def etri_body(p, c2):
                dls = [eload(3 * p + k, k) for k in range(3)]
                scs = []
                for k in range(3):
                    dls[k][0].wait()
                    dls[k][1].wait()
                    scs.append(edge_compute(k))
                for s in scs:
                    s.wait()
                return c2
            lax.fori_loop(0, SB // 3, etri_body, 0)
            dw, dr = eload(SB - 1, 0)
            dw.wait()
            dr.wait()
            edge_compute(0).wait()
            return carry
        lax.fori_loop(0, ept // SE, edge_super, 0)
        plsc.subcore_barrier()

        pltpu.sync_copy(hacc_sh.at[pl.ds(sid * rpt, rpt)],
                        out_hbm.at[cid, pl.ds(sid * rpt, rpt)])

    return sc_kernel


def kernel(x, edge_index, edge_type, batch_size, W_rel, W_self, b, W_lin, b_lin):
    N, D = x.shape
    R, _, H = W_rel.shape
    E = edge_index.shape[1]
    B = 16

    Wcat = jnp.concatenate([W_rel, W_self[None]], axis=0)
    Y = _project_all(x, Wcat)
    Yflat = Y.reshape((R + 1) * N, H)

    src = edge_index[0].astype(jnp.int32)
    dst = edge_index[1].astype(jnp.int32)
    et = edge_type.astype(jnp.int32)

    sc = _make_sc_edge_kernel(N, E, H, R)
    kslice = -(-(N * R) // (_NS * 8)) * 8
    rpt = -(-(N // _NS) // 8) * 8
    zc = jnp.zeros((kslice,), jnp.float32)
    zr = jnp.zeros((rpt, H), jnp.float32)
    P = sc(Yflat, src, dst, et, zc, zr)[:, :N, :]

    h0r = Y[R].reshape(B, N // B, H)
    Pr = P.reshape(2, B, N // B, H)
    osum = _pool_head(h0r, Pr, b, W_lin)
    per_graph = N // batch_size
    out = osum / per_graph.astype(jnp.float32) + b_lin[None, :]
    return jnp.squeeze(out)
